# baseline (speedup 1.0000x reference)
import jax
import jax.numpy as jnp
from jax import lax
from jax.experimental import pallas as pl
from jax.experimental.pallas import tpu as pltpu
from jax.experimental.pallas import tpu_sc as plsc

N = 100000
D = 32
F = D // 2
E = 1600000
NC, NS = 2, 16
NP = 100096
NPQ = NP // 4
TRASH = 512
SUB = 4
CHUNK = SUB * 128
E_PAD = 1638400
STEPS = E_PAD // NS // CHUNK
NIB = 4
_UNROLL = 4
RPT = NP // NS


ZR = 391


def _segsum_body(x_hbm, sd_hbm, out_hbm, idxb, rows, acc, zbuf,
                 isem0, isem1, isem2, isem3, gsem0, gsem1, ssem0, ssem1):
    isem = (isem0, isem1, isem2, isem3)
    gsem = (gsem0, gsem1)
    ssem = (ssem0, ssem1)
    c = lax.axis_index("c")
    s = lax.axis_index("s")
    row0 = s * (E_PAD // NS // 128)

    def zrow(r, _):
        zbuf[r, :] = jnp.zeros((16,), jnp.float32)
        return ()

    lax.fori_loop(0, ZR, zrow, (), unroll=False)
    for k in range(RPT // ZR):
        pltpu.sync_copy(zbuf, acc.at[pl.ds(s * RPT + k * ZR, ZR)])
    plsc.subcore_barrier()

    def fire_idx(i, q):
        for t in range(2):
            pltpu.async_copy(sd_hbm.at[c, t, pl.ds(row0 + i * SUB, SUB)],
                             idxb.at[q, t], isem[q])

    def wait_idx(q):
        for t in range(2):
            pltpu.make_async_copy(sd_hbm.at[c, t, pl.ds(0, SUB)],
                                  idxb.at[q, t], isem[q]).wait()

    def fire_gathers(q, rb):
        for j in range(SUB):
            pltpu.async_copy(x_hbm.at[idxb.at[q, 0, j]],
                             rows.at[rb, pl.ds(j * 128, 128)], gsem[rb])

    def wait_gathers(q, rb):
        for j in range(SUB):
            pltpu.make_async_copy(x_hbm.at[idxb.at[q, 0, j]],
                                  rows.at[rb, pl.ds(j * 128, 128)],
                                  gsem[rb]).wait()

    def fire_scatters(q, rb):
        for j in range(SUB):
            pltpu.async_copy(rows.at[rb, pl.ds(j * 128, 128)],
                             acc.at[idxb.at[q, 1, j]], ssem[rb], add=True)

    def wait_scatters(q, rb):
        for j in range(SUB):
            pltpu.make_async_copy(rows.at[rb, pl.ds(j * 128, 128)],
                                  acc.at[idxb.at[q, 1, j]], ssem[rb]).wait()

    fire_idx(0, 0)
    fire_idx(1, 1)
    wait_idx(0)
    fire_gathers(0, 0)

    def iter4(i2, _):
        for u in range(_UNROLL):
            i = i2 * _UNROLL + u
            q, rb = u % NIB, u % 2
            qn, rbn = (u + 1) % NIB, (u + 1) % 2
            qp = (u - 1) % NIB
            if u < 2:
                fire_idx(i + 2, (u + 2) % NIB)
            else:
                @pl.when(i2 < STEPS // _UNROLL - 1)
                def _():
                    fire_idx(i + 2, (u + 2) % NIB)
            def prep():
                wait_idx(qn)
                fire_gathers(qn, rbn)
            if u == 0:
                @pl.when(i2 >= 1)
                def _():
                    wait_scatters(qp, rbn)
                prep()
            elif u < _UNROLL - 1:
                wait_scatters(qp, rbn)
                prep()
            else:
                @pl.when(i2 < STEPS // _UNROLL - 1)
                def _():
                    wait_scatters(qp, rbn)
                    prep()
            wait_gathers(q, rb)
            fire_scatters(q, rb)
        return ()

    lax.fori_loop(0, STEPS // _UNROLL, iter4, (), unroll=False)
    wait_scatters((STEPS - 2) % NIB, 0)
    wait_scatters((STEPS - 1) % NIB, 1)
    plsc.subcore_barrier()

    pltpu.sync_copy(
        acc.at[pl.ds(s * RPT, RPT)],
        out_hbm.at[pl.ds(s * RPT, RPT), c],
    )


_segsum = pl.kernel(
    _segsum_body,
    out_type=jax.ShapeDtypeStruct((NP, 2, F), jnp.float32),
    mesh=plsc.VectorSubcoreMesh(core_axis_name="c", subcore_axis_name="s"),
    scratch_types=[
        pltpu.VMEM((NIB, 2, SUB, 128), jnp.int32),
        pltpu.VMEM((2, CHUNK, F), jnp.float32),
        pltpu.VMEM_SHARED((NP + TRASH, F), jnp.float32),
        pltpu.VMEM((ZR, F), jnp.float32),
        pltpu.SemaphoreType.DMA,
        pltpu.SemaphoreType.DMA,
        pltpu.SemaphoreType.DMA,
        pltpu.SemaphoreType.DMA,
        pltpu.SemaphoreType.DMA,
        pltpu.SemaphoreType.DMA,
        pltpu.SemaphoreType.DMA,
        pltpu.SemaphoreType.DMA,
    ],
    compiler_params=pltpu.CompilerParams(use_tc_tiling_on_sc=False),
)


def _elu(v):
    return jnp.where(v > 0.0, v, jnp.exp(jnp.minimum(v, 0.0)) - 1.0)


def _branch(h, w1, b1, w2, b2, lw, lb):
    t = _elu(jnp.dot(h, w1, preferred_element_type=jnp.float32) + b1)
    t = _elu(jnp.dot(t, w2, preferred_element_type=jnp.float32) + b2)
    return _elu(jnp.dot(t, lw, preferred_element_type=jnp.float32) + lb)


RQ = 2048
_GRID = (pl.cdiv(NPQ, RQ),)
_rq = pl.BlockSpec((RQ, 128), lambda i: (i, 0))


def _full(shape):
    return pl.BlockSpec(shape, lambda i: (0,) * len(shape))


def _layer1_body(x, a0, a1, w1, b1, w2, b2, lw, lb, out):
    acc = None
    for j in range(2):
        h = x[...] + (a0[...] if j == 0 else a1[...])
        t = _branch(h, w1[j], b1[j], w2[j], b2[j], lw[j], lb[j])
        acc = t if acc is None else acc + t
    out[...] = acc


_layer1 = pl.pallas_call(
    _layer1_body,
    grid=_GRID,
    in_specs=[
        _rq, _rq, _rq,
        _full((2, 128, 128)), _full((2, 128)), _full((2, 128, 128)),
        _full((2, 128)), _full((2, 128, 128)), _full((2, 128)),
    ],
    out_specs=_rq,
    out_shape=jax.ShapeDtypeStruct((NPQ, 128), jnp.float32),
)


def _layer2_body(x1, a0, a1, w1, b1, w2, b2, lw, lb, wla, wlb, bl, out):
    acc = None
    for j in range(2):
        h = x1[...] + (a0[...] if j == 0 else a1[...])
        t = _branch(h, w1[j], b1[j], w2[j], b2[j], lw[j], lb[j])
        acc = t if acc is None else acc + t
    out[...] = (
        jnp.dot(x1[...], wla[...], preferred_element_type=jnp.float32)
        + jnp.dot(acc, wlb[...], preferred_element_type=jnp.float32)
        + bl[...]
    )


_layer2 = pl.pallas_call(
    _layer2_body,
    grid=_GRID,
    in_specs=[
        _rq, _rq, _rq,
        _full((2, 128, 128)), _full((2, 128)), _full((2, 128, 128)),
        _full((2, 128)), _full((2, 128, 128)), _full((2, 128)),
        _full((128, 128)), _full((128, 128)), _full((128,)),
    ],
    out_specs=_rq,
    out_shape=jax.ShapeDtypeStruct((NPQ, 128), jnp.float32),
)


def _bd(w):
    return jnp.kron(jnp.eye(4, dtype=w.dtype), w)


def _bd2(w):
    return jnp.stack([_bd(w[0]), _bd(w[1])])


def _t4(b):
    return jnp.tile(b, (1, 4))


def _prep(adj):
    pad = E_PAD - E
    pidx = jnp.arange(pad, dtype=jnp.int32)
    s2 = 2 * jnp.concatenate([adj[0], pidx % N])
    dst = jnp.concatenate([adj[1], NP + (pidx & (TRASH - 1))])
    return jnp.stack([jnp.stack([s2, dst]),
                      jnp.stack([s2 + 1, dst])]).reshape(
                          NC, 2, E_PAD // 128, 128)


def kernel(adjs_0, adjs_1, embed, gin_w1, gin_b1, gin_w2, gin_b2, lin_w, lin_b, w_last, b_last):
    sd0 = _prep(adjs_0)
    sd1 = _prep(adjs_1)
    x0 = jnp.pad(embed, ((0, NP - N), (0, 0)))

    def segsum(x, sd):
        return _segsum(x.reshape(2 * NP, F), sd).reshape(NPQ, 128)

    w1a, b1a = _bd2(gin_w1[0]), _t4(gin_b1[0])
    w2a, b2a = _bd2(gin_w2[0]), _t4(gin_b2[0])
    lwa, lba = _bd2(lin_w[0]), _t4(lin_b[0])
    w1b, b1b = _bd2(gin_w1[1]), _t4(gin_b1[1])
    w2b, b2b = _bd2(gin_w2[1]), _t4(gin_b2[1])
    lwb, lbb = _bd2(lin_w[1]), _t4(lin_b[1])
    wla, wlb = _bd(w_last[:D]), _bd(w_last[D:])
    blt = jnp.tile(b_last, 4)

    x0q = x0.reshape(NPQ, 128)
    hp0 = segsum(x0, sd0)
    hp1 = segsum(x0, sd1)
    x1 = _layer1(x0q, hp0, hp1, w1a, b1a, w2a, b2a, lwa, lba)
    x1n = x1.reshape(NP, D)
    hp0b = segsum(x1n, sd0)
    hp1b = segsum(x1n, sd1)
    y = _layer2(x1, hp0b, hp1b, w1b, b1b, w2b, b2b, lwb, lbb, wla, wlb, blt)
    return y.reshape(NP, D)[:N]


# speedup vs baseline: 1.0783x; 1.0783x over previous
import jax
import jax.numpy as jnp
from jax import lax
from jax.experimental import pallas as pl
from jax.experimental.pallas import tpu as pltpu
from jax.experimental.pallas import tpu_sc as plsc

N = 100000
D = 32
F = D // 2
E = 1600000
NC, NS = 2, 16
NP = 100096
NPQ = NP // 4
TRASH = 512
SUB = 4
CHUNK = SUB * 128
E_PAD = 1638400
STEPS = E_PAD // NS // CHUNK
NIB = 4
_UNROLL = 4
RPT = NP // NS


ZR = 391


def _segsum_body(x_hbm, sd_hbm, out_hbm, idxb, rows, acc, zbuf,
                 isem0, isem1, isem2, isem3, gsem0, gsem1, ssem0, ssem1):
    isem = (isem0, isem1, isem2, isem3)
    gsem = (gsem0, gsem1)
    ssem = (ssem0, ssem1)
    c = lax.axis_index("c")
    s = lax.axis_index("s")
    row0 = s * (E_PAD // NS // 128)

    def zrow(r, _):
        zbuf[r, :] = jnp.zeros((16,), jnp.float32)
        return ()

    lax.fori_loop(0, ZR, zrow, (), unroll=False)
    for k in range(RPT // ZR):
        pltpu.sync_copy(zbuf, acc.at[pl.ds(s * RPT + k * ZR, ZR)])
    plsc.subcore_barrier()

    def fire_idx(i, q):
        for t in range(2):
            pltpu.async_copy(sd_hbm.at[c, t, pl.ds(row0 + i * SUB, SUB)],
                             idxb.at[q, t], isem[q])

    def wait_idx(q):
        for t in range(2):
            pltpu.make_async_copy(sd_hbm.at[c, t, pl.ds(0, SUB)],
                                  idxb.at[q, t], isem[q]).wait()

    def fire_gathers(q, rb):
        for j in range(SUB):
            pltpu.async_copy(x_hbm.at[idxb.at[q, 0, j]],
                             rows.at[rb, pl.ds(j * 128, 128)], gsem[rb])

    def wait_gathers(q, rb):
        for j in range(SUB):
            pltpu.make_async_copy(x_hbm.at[idxb.at[q, 0, j]],
                                  rows.at[rb, pl.ds(j * 128, 128)],
                                  gsem[rb]).wait()

    def fire_scatters(q, rb):
        for j in range(SUB):
            pltpu.async_copy(rows.at[rb, pl.ds(j * 128, 128)],
                             acc.at[idxb.at[q, 1, j]], ssem[rb], add=True)

    def wait_scatters(q, rb):
        for j in range(SUB):
            pltpu.make_async_copy(rows.at[rb, pl.ds(j * 128, 128)],
                                  acc.at[idxb.at[q, 1, j]], ssem[rb]).wait()

    fire_idx(0, 0)
    fire_idx(1, 1)
    wait_idx(0)
    fire_gathers(0, 0)

    def iter4(i2, _):
        for u in range(_UNROLL):
            i = i2 * _UNROLL + u
            q, rb = u % NIB, u % 2
            qn, rbn = (u + 1) % NIB, (u + 1) % 2
            qp = (u - 1) % NIB
            if u < 2:
                fire_idx(i + 2, (u + 2) % NIB)
            else:
                @pl.when(i2 < STEPS // _UNROLL - 1)
                def _():
                    fire_idx(i + 2, (u + 2) % NIB)
            def prep():
                wait_idx(qn)
                fire_gathers(qn, rbn)
            if u == 0:
                @pl.when(i2 >= 1)
                def _():
                    wait_scatters(qp, rbn)
                prep()
            elif u < _UNROLL - 1:
                wait_scatters(qp, rbn)
                prep()
            else:
                @pl.when(i2 < STEPS // _UNROLL - 1)
                def _():
                    wait_scatters(qp, rbn)
                    prep()
            wait_gathers(q, rb)
            fire_scatters(q, rb)
        return ()

    lax.fori_loop(0, STEPS // _UNROLL, iter4, (), unroll=False)
    wait_scatters((STEPS - 2) % NIB, 0)
    wait_scatters((STEPS - 1) % NIB, 1)
    plsc.subcore_barrier()

    pltpu.sync_copy(
        acc.at[pl.ds(s * RPT, RPT)],
        out_hbm.at[pl.ds(s * RPT, RPT), c],
    )


_segsum = pl.kernel(
    _segsum_body,
    out_type=jax.ShapeDtypeStruct((NP, 2, F), jnp.float32),
    mesh=plsc.VectorSubcoreMesh(core_axis_name="c", subcore_axis_name="s"),
    scratch_types=[
        pltpu.VMEM((NIB, 2, SUB, 128), jnp.int32),
        pltpu.VMEM((2, CHUNK, F), jnp.float32),
        pltpu.VMEM_SHARED((NP + TRASH, F), jnp.float32),
        pltpu.VMEM((ZR, F), jnp.float32),
        pltpu.SemaphoreType.DMA,
        pltpu.SemaphoreType.DMA,
        pltpu.SemaphoreType.DMA,
        pltpu.SemaphoreType.DMA,
        pltpu.SemaphoreType.DMA,
        pltpu.SemaphoreType.DMA,
        pltpu.SemaphoreType.DMA,
        pltpu.SemaphoreType.DMA,
    ],
    compiler_params=pltpu.CompilerParams(use_tc_tiling_on_sc=False),
)


def _elu(v):
    return jnp.where(v > 0.0, v, jnp.exp(jnp.minimum(v, 0.0)) - 1.0)


def _branch(h, w1, b1, w2, b2, lw, lb):
    t = _elu(jnp.dot(h, w1, preferred_element_type=jnp.float32) + b1)
    t = _elu(jnp.dot(t, w2, preferred_element_type=jnp.float32) + b2)
    return _elu(jnp.dot(t, lw, preferred_element_type=jnp.float32) + lb)


RQ = 2048
_GRID = (pl.cdiv(NPQ, RQ),)
_rq = pl.BlockSpec((RQ, 128), lambda i: (i, 0))


def _full(shape):
    return pl.BlockSpec(shape, lambda i: (0,) * len(shape))


def _layer1_body(x, a0, a1, w1, b1, w2, b2, lw, lb, out):
    acc = None
    for j in range(2):
        h = x[...] + (a0[...] if j == 0 else a1[...])
        t = _branch(h, w1[j], b1[j], w2[j], b2[j], lw[j], lb[j])
        acc = t if acc is None else acc + t
    out[...] = acc


_layer1 = pl.pallas_call(
    _layer1_body,
    grid=_GRID,
    in_specs=[
        _rq, _rq, _rq,
        _full((2, 128, 128)), _full((2, 128)), _full((2, 128, 128)),
        _full((2, 128)), _full((2, 128, 128)), _full((2, 128)),
    ],
    out_specs=_rq,
    out_shape=jax.ShapeDtypeStruct((NPQ, 128), jnp.float32),
)


def _layer2_body(x1, a0, a1, w1, b1, w2, b2, lw, lb, wla, wlb, bl, out):
    acc = None
    for j in range(2):
        h = x1[...] + (a0[...] if j == 0 else a1[...])
        t = _branch(h, w1[j], b1[j], w2[j], b2[j], lw[j], lb[j])
        acc = t if acc is None else acc + t
    out[...] = (
        jnp.dot(x1[...], wla[...], preferred_element_type=jnp.float32)
        + jnp.dot(acc, wlb[...], preferred_element_type=jnp.float32)
        + bl[...]
    )


_layer2 = pl.pallas_call(
    _layer2_body,
    grid=_GRID,
    in_specs=[
        _rq, _rq, _rq,
        _full((2, 128, 128)), _full((2, 128)), _full((2, 128, 128)),
        _full((2, 128)), _full((2, 128, 128)), _full((2, 128)),
        _full((128, 128)), _full((128, 128)), _full((128,)),
    ],
    out_specs=_rq,
    out_shape=jax.ShapeDtypeStruct((NPQ, 128), jnp.float32),
)


def _bd(w):
    return jnp.kron(jnp.eye(4, dtype=w.dtype), w)


def _bd2(w):
    return jnp.stack([_bd(w[0]), _bd(w[1])])


def _t4(b):
    return jnp.tile(b, (1, 4))


def _prep(adj):
    pad = E_PAD - E
    pidx = jnp.arange(pad, dtype=jnp.int32)
    s2 = 2 * jnp.concatenate([adj[0], pidx % N])
    dst = jnp.concatenate([adj[1], NP + (pidx & (TRASH - 1))])
    return jnp.stack([jnp.stack([s2, dst]),
                      jnp.stack([s2 + 1, dst])]).reshape(
                          NC, 2, E_PAD // 128, 128)


def kernel(adjs_0, adjs_1, embed, gin_w1, gin_b1, gin_w2, gin_b2, lin_w, lin_b, w_last, b_last):
    x0 = jnp.pad(embed, ((0, NP - N), (0, 0)))
    x0f = x0.reshape(2 * NP, F)
    adjs_0 = lax.optimization_barrier((adjs_0, x0f))[0]
    sd0 = _prep(adjs_0)
    adjs_1 = lax.optimization_barrier((adjs_1, sd0))[0]
    sd1 = _prep(adjs_1)

    def segsum(x, sd):
        return _segsum(x.reshape(2 * NP, F), sd).reshape(NPQ, 128)

    w1a, b1a = _bd2(gin_w1[0]), _t4(gin_b1[0])
    w2a, b2a = _bd2(gin_w2[0]), _t4(gin_b2[0])
    lwa, lba = _bd2(lin_w[0]), _t4(lin_b[0])
    w1b, b1b = _bd2(gin_w1[1]), _t4(gin_b1[1])
    w2b, b2b = _bd2(gin_w2[1]), _t4(gin_b2[1])
    lwb, lbb = _bd2(lin_w[1]), _t4(lin_b[1])
    wla, wlb = _bd(w_last[:D]), _bd(w_last[D:])
    blt = jnp.tile(b_last, 4)

    x0q = x0.reshape(NPQ, 128)
    hp0 = segsum(x0, sd0)
    hp1 = segsum(x0, sd1)
    x1 = _layer1(x0q, hp0, hp1, w1a, b1a, w2a, b2a, lwa, lba)
    x1n = x1.reshape(NP, D)
    hp0b = segsum(x1n, sd0)
    hp1b = segsum(x1n, sd1)
    y = _layer2(x1, hp0b, hp1b, w1b, b1b, w2b, b2b, lwb, lbb, wla, wlb, blt)
    return y.reshape(NP, D)[:N]


# speedup vs baseline: 1.0807x; 1.0022x over previous
import jax
import jax.numpy as jnp
from jax import lax
from jax.experimental import pallas as pl
from jax.experimental.pallas import tpu as pltpu
from jax.experimental.pallas import tpu_sc as plsc

N = 100000
D = 32
F = D // 2
E = 1600000
NC, NS = 2, 16
NP = 100096
NPQ = NP // 4
TRASH = 512
SUB = 4
CHUNK = SUB * 128
E_PAD = 1638400
STEPS = E_PAD // NS // CHUNK
NIB = 4
_UNROLL = 4
RPT = NP // NS


ZR = 391


def _segsum_body(x_hbm, sd_hbm, out_hbm, idxb, rows, acc, zbuf,
                 isem0, isem1, isem2, isem3, gsem0, gsem1, ssem0, ssem1):
    isem = (isem0, isem1, isem2, isem3)
    gsem = (gsem0, gsem1)
    ssem = (ssem0, ssem1)
    c = lax.axis_index("c")
    s = lax.axis_index("s")
    row0 = s * (E_PAD // NS // 128)

    def fire_idx(i, q):
        for t in range(2):
            pltpu.async_copy(sd_hbm.at[c, t, pl.ds(row0 + i * SUB, SUB)],
                             idxb.at[q, t], isem[q])

    def wait_idx(q):
        for t in range(2):
            pltpu.make_async_copy(sd_hbm.at[c, t, pl.ds(0, SUB)],
                                  idxb.at[q, t], isem[q]).wait()

    def fire_gathers(q, rb):
        for j in range(SUB):
            pltpu.async_copy(x_hbm.at[idxb.at[q, 0, j]],
                             rows.at[rb, pl.ds(j * 128, 128)], gsem[rb])

    def wait_gathers(q, rb):
        for j in range(SUB):
            pltpu.make_async_copy(x_hbm.at[idxb.at[q, 0, j]],
                                  rows.at[rb, pl.ds(j * 128, 128)],
                                  gsem[rb]).wait()

    def fire_scatters(q, rb):
        for j in range(SUB):
            pltpu.async_copy(rows.at[rb, pl.ds(j * 128, 128)],
                             acc.at[idxb.at[q, 1, j]], ssem[rb], add=True)

    def wait_scatters(q, rb):
        for j in range(SUB):
            pltpu.make_async_copy(rows.at[rb, pl.ds(j * 128, 128)],
                                  acc.at[idxb.at[q, 1, j]], ssem[rb]).wait()

    fire_idx(0, 0)
    fire_idx(1, 1)
    wait_idx(0)
    fire_gathers(0, 0)

    def zrow(r, _):
        zbuf[r, :] = jnp.zeros((16,), jnp.float32)
        return ()

    lax.fori_loop(0, ZR, zrow, (), unroll=False)
    for k in range(RPT // ZR):
        pltpu.sync_copy(zbuf, acc.at[pl.ds(s * RPT + k * ZR, ZR)])
    plsc.subcore_barrier()

    def iter4(i2, _):
        for u in range(_UNROLL):
            i = i2 * _UNROLL + u
            q, rb = u % NIB, u % 2
            qn, rbn = (u + 1) % NIB, (u + 1) % 2
            qp = (u - 1) % NIB
            if u < 2:
                fire_idx(i + 2, (u + 2) % NIB)
            else:
                @pl.when(i2 < STEPS // _UNROLL - 1)
                def _():
                    fire_idx(i + 2, (u + 2) % NIB)
            def prep():
                wait_idx(qn)
                fire_gathers(qn, rbn)
            if u == 0:
                @pl.when(i2 >= 1)
                def _():
                    wait_scatters(qp, rbn)
                prep()
            elif u < _UNROLL - 1:
                wait_scatters(qp, rbn)
                prep()
            else:
                @pl.when(i2 < STEPS // _UNROLL - 1)
                def _():
                    wait_scatters(qp, rbn)
                    prep()
            wait_gathers(q, rb)
            fire_scatters(q, rb)
        return ()

    lax.fori_loop(0, STEPS // _UNROLL, iter4, (), unroll=False)
    wait_scatters((STEPS - 2) % NIB, 0)
    wait_scatters((STEPS - 1) % NIB, 1)
    plsc.subcore_barrier()

    pltpu.sync_copy(
        acc.at[pl.ds(s * RPT, RPT)],
        out_hbm.at[pl.ds(s * RPT, RPT), c],
    )


_segsum = pl.kernel(
    _segsum_body,
    out_type=jax.ShapeDtypeStruct((NP, 2, F), jnp.float32),
    mesh=plsc.VectorSubcoreMesh(core_axis_name="c", subcore_axis_name="s"),
    scratch_types=[
        pltpu.VMEM((NIB, 2, SUB, 128), jnp.int32),
        pltpu.VMEM((2, CHUNK, F), jnp.float32),
        pltpu.VMEM_SHARED((NP + TRASH, F), jnp.float32),
        pltpu.VMEM((ZR, F), jnp.float32),
        pltpu.SemaphoreType.DMA,
        pltpu.SemaphoreType.DMA,
        pltpu.SemaphoreType.DMA,
        pltpu.SemaphoreType.DMA,
        pltpu.SemaphoreType.DMA,
        pltpu.SemaphoreType.DMA,
        pltpu.SemaphoreType.DMA,
        pltpu.SemaphoreType.DMA,
    ],
    compiler_params=pltpu.CompilerParams(use_tc_tiling_on_sc=False),
)


def _elu(v):
    return jnp.where(v > 0.0, v, jnp.exp(jnp.minimum(v, 0.0)) - 1.0)


def _branch(h, w1, b1, w2, b2, lw, lb):
    t = _elu(jnp.dot(h, w1, preferred_element_type=jnp.float32) + b1)
    t = _elu(jnp.dot(t, w2, preferred_element_type=jnp.float32) + b2)
    return _elu(jnp.dot(t, lw, preferred_element_type=jnp.float32) + lb)


RQ = 2048
_GRID = (pl.cdiv(NPQ, RQ),)
_rq = pl.BlockSpec((RQ, 128), lambda i: (i, 0))


def _full(shape):
    return pl.BlockSpec(shape, lambda i: (0,) * len(shape))


def _layer1_body(x, a0, a1, w1, b1, w2, b2, lw, lb, out):
    acc = None
    for j in range(2):
        h = x[...] + (a0[...] if j == 0 else a1[...])
        t = _branch(h, w1[j], b1[j], w2[j], b2[j], lw[j], lb[j])
        acc = t if acc is None else acc + t
    out[...] = acc


_layer1 = pl.pallas_call(
    _layer1_body,
    grid=_GRID,
    in_specs=[
        _rq, _rq, _rq,
        _full((2, 128, 128)), _full((2, 128)), _full((2, 128, 128)),
        _full((2, 128)), _full((2, 128, 128)), _full((2, 128)),
    ],
    out_specs=_rq,
    out_shape=jax.ShapeDtypeStruct((NPQ, 128), jnp.float32),
)


def _layer2_body(x1, a0, a1, w1, b1, w2, b2, lw, lb, wla, wlb, bl, out):
    acc = None
    for j in range(2):
        h = x1[...] + (a0[...] if j == 0 else a1[...])
        t = _branch(h, w1[j], b1[j], w2[j], b2[j], lw[j], lb[j])
        acc = t if acc is None else acc + t
    out[...] = (
        jnp.dot(x1[...], wla[...], preferred_element_type=jnp.float32)
        + jnp.dot(acc, wlb[...], preferred_element_type=jnp.float32)
        + bl[...]
    )


_layer2 = pl.pallas_call(
    _layer2_body,
    grid=_GRID,
    in_specs=[
        _rq, _rq, _rq,
        _full((2, 128, 128)), _full((2, 128)), _full((2, 128, 128)),
        _full((2, 128)), _full((2, 128, 128)), _full((2, 128)),
        _full((128, 128)), _full((128, 128)), _full((128,)),
    ],
    out_specs=_rq,
    out_shape=jax.ShapeDtypeStruct((NPQ, 128), jnp.float32),
)


def _bd(w):
    return jnp.kron(jnp.eye(4, dtype=w.dtype), w)


def _bd2(w):
    return jnp.stack([_bd(w[0]), _bd(w[1])])


def _t4(b):
    return jnp.tile(b, (1, 4))


def _prep(adj):
    pad = E_PAD - E
    pidx = jnp.arange(pad, dtype=jnp.int32)
    s2 = 2 * jnp.concatenate([adj[0], pidx % N])
    dst = jnp.concatenate([adj[1], NP + (pidx & (TRASH - 1))])
    return jnp.stack([jnp.stack([s2, dst]),
                      jnp.stack([s2 + 1, dst])]).reshape(
                          NC, 2, E_PAD // 128, 128)


def kernel(adjs_0, adjs_1, embed, gin_w1, gin_b1, gin_w2, gin_b2, lin_w, lin_b, w_last, b_last):
    x0 = jnp.pad(embed, ((0, NP - N), (0, 0)))
    x0f = x0.reshape(2 * NP, F)
    adjs_0 = lax.optimization_barrier((adjs_0, x0f))[0]
    sd0 = _prep(adjs_0)
    adjs_1 = lax.optimization_barrier((adjs_1, sd0))[0]
    sd1 = _prep(adjs_1)

    def segsum(x, sd):
        return _segsum(x.reshape(2 * NP, F), sd).reshape(NPQ, 128)

    w1a, b1a = _bd2(gin_w1[0]), _t4(gin_b1[0])
    w2a, b2a = _bd2(gin_w2[0]), _t4(gin_b2[0])
    lwa, lba = _bd2(lin_w[0]), _t4(lin_b[0])
    w1b, b1b = _bd2(gin_w1[1]), _t4(gin_b1[1])
    w2b, b2b = _bd2(gin_w2[1]), _t4(gin_b2[1])
    lwb, lbb = _bd2(lin_w[1]), _t4(lin_b[1])
    wla, wlb = _bd(w_last[:D]), _bd(w_last[D:])
    blt = jnp.tile(b_last, 4)

    x0q = x0.reshape(NPQ, 128)
    hp0 = segsum(x0, sd0)
    hp1 = segsum(x0, sd1)
    x1 = _layer1(x0q, hp0, hp1, w1a, b1a, w2a, b2a, lwa, lba)
    x1n = x1.reshape(NP, D)
    hp0b = segsum(x1n, sd0)
    hp1b = segsum(x1n, sd1)
    y = _layer2(x1, hp0b, hp1b, w1b, b1b, w2b, b2b, lwb, lbb, wla, wlb, blt)
    return y.reshape(NP, D)[:N]


# speedup vs baseline: 1.1310x; 1.0465x over previous
import jax
import jax.numpy as jnp
from jax import lax
from jax.experimental import pallas as pl
from jax.experimental.pallas import tpu as pltpu
from jax.experimental.pallas import tpu_sc as plsc

N = 100000
D = 32
F = D // 2
E = 1600000
NC, NS = 2, 16
NP = 100096
NPQ = NP // 4
TRASH = 64
SUB = 5
CHUNK = SUB * 128
E_PAD = 1638400
STEPS = E_PAD // NS // CHUNK
NIB = 4
_UNROLL = 4
RPT = NP // NS


ZR = 184


def _segsum_body(x_hbm, sd_hbm, out_hbm, idxb, rows, acc, zbuf,
                 isem0, isem1, isem2, isem3, gsem0, gsem1, ssem0, ssem1):
    isem = (isem0, isem1, isem2, isem3)
    gsem = (gsem0, gsem1)
    ssem = (ssem0, ssem1)
    c = lax.axis_index("c")
    s = lax.axis_index("s")
    row0 = s * (E_PAD // NS // 128)

    def fire_idx(i, q):
        for t in range(2):
            pltpu.async_copy(sd_hbm.at[c, t, pl.ds(row0 + i * SUB, SUB)],
                             idxb.at[q, t], isem[q])

    def wait_idx(q):
        for t in range(2):
            pltpu.make_async_copy(sd_hbm.at[c, t, pl.ds(0, SUB)],
                                  idxb.at[q, t], isem[q]).wait()

    def fire_gathers(q, rb):
        for j in range(SUB):
            pltpu.async_copy(x_hbm.at[idxb.at[q, 0, j]],
                             rows.at[rb, pl.ds(j * 128, 128)], gsem[rb])

    def wait_gathers(q, rb):
        for j in range(SUB):
            pltpu.make_async_copy(x_hbm.at[idxb.at[q, 0, j]],
                                  rows.at[rb, pl.ds(j * 128, 128)],
                                  gsem[rb]).wait()

    def fire_scatters(q, rb):
        for j in range(SUB):
            pltpu.async_copy(rows.at[rb, pl.ds(j * 128, 128)],
                             acc.at[idxb.at[q, 1, j]], ssem[rb], add=True)

    def wait_scatters(q, rb):
        for j in range(SUB):
            pltpu.make_async_copy(rows.at[rb, pl.ds(j * 128, 128)],
                                  acc.at[idxb.at[q, 1, j]], ssem[rb]).wait()

    fire_idx(0, 0)
    fire_idx(1, 1)
    wait_idx(0)
    fire_gathers(0, 0)

    def zrow(r, _):
        zbuf[r, :] = jnp.zeros((16,), jnp.float32)
        return ()

    lax.fori_loop(0, ZR, zrow, (), unroll=False)
    for k in range(RPT // ZR):
        pltpu.sync_copy(zbuf, acc.at[pl.ds(s * RPT + k * ZR, ZR)])
    plsc.subcore_barrier()

    def iter4(i2, _):
        for u in range(_UNROLL):
            i = i2 * _UNROLL + u
            q, rb = u % NIB, u % 2
            qn, rbn = (u + 1) % NIB, (u + 1) % 2
            qp = (u - 1) % NIB
            if u < 2:
                fire_idx(i + 2, (u + 2) % NIB)
            else:
                @pl.when(i2 < STEPS // _UNROLL - 1)
                def _():
                    fire_idx(i + 2, (u + 2) % NIB)
            def prep():
                wait_idx(qn)
                fire_gathers(qn, rbn)
            if u == 0:
                @pl.when(i2 >= 1)
                def _():
                    wait_scatters(qp, rbn)
                prep()
            elif u < _UNROLL - 1:
                wait_scatters(qp, rbn)
                prep()
            else:
                @pl.when(i2 < STEPS // _UNROLL - 1)
                def _():
                    wait_scatters(qp, rbn)
                    prep()
            wait_gathers(q, rb)
            fire_scatters(q, rb)
        return ()

    lax.fori_loop(0, STEPS // _UNROLL, iter4, (), unroll=False)
    wait_scatters((STEPS - 2) % NIB, 0)
    wait_scatters((STEPS - 1) % NIB, 1)
    plsc.subcore_barrier()

    pltpu.sync_copy(
        acc.at[pl.ds(s * RPT, RPT)],
        out_hbm.at[pl.ds(s * RPT, RPT), c],
    )


_segsum = pl.kernel(
    _segsum_body,
    out_type=jax.ShapeDtypeStruct((NP, 2, F), jnp.float32),
    mesh=plsc.VectorSubcoreMesh(core_axis_name="c", subcore_axis_name="s"),
    scratch_types=[
        pltpu.VMEM((NIB, 2, SUB, 128), jnp.int32),
        pltpu.VMEM((2, CHUNK, F), jnp.float32),
        pltpu.VMEM_SHARED((NP + TRASH, F), jnp.float32),
        pltpu.VMEM((ZR, F), jnp.float32),
        pltpu.SemaphoreType.DMA,
        pltpu.SemaphoreType.DMA,
        pltpu.SemaphoreType.DMA,
        pltpu.SemaphoreType.DMA,
        pltpu.SemaphoreType.DMA,
        pltpu.SemaphoreType.DMA,
        pltpu.SemaphoreType.DMA,
        pltpu.SemaphoreType.DMA,
    ],
    compiler_params=pltpu.CompilerParams(use_tc_tiling_on_sc=False),
)


def _elu(v):
    return jnp.where(v > 0.0, v, jnp.exp(jnp.minimum(v, 0.0)) - 1.0)


def _branch(h, w1, b1, w2, b2, lw, lb):
    t = _elu(jnp.dot(h, w1, preferred_element_type=jnp.float32) + b1)
    t = _elu(jnp.dot(t, w2, preferred_element_type=jnp.float32) + b2)
    return _elu(jnp.dot(t, lw, preferred_element_type=jnp.float32) + lb)


RQ = 2048
_GRID = (pl.cdiv(NPQ, RQ),)
_rq = pl.BlockSpec((RQ, 128), lambda i: (i, 0))


def _full(shape):
    return pl.BlockSpec(shape, lambda i: (0,) * len(shape))


def _layer1_body(x, a0, a1, w1, b1, w2, b2, lw, lb, out):
    acc = None
    for j in range(2):
        h = x[...] + (a0[...] if j == 0 else a1[...])
        t = _branch(h, w1[j], b1[j], w2[j], b2[j], lw[j], lb[j])
        acc = t if acc is None else acc + t
    out[...] = acc


_layer1 = pl.pallas_call(
    _layer1_body,
    grid=_GRID,
    in_specs=[
        _rq, _rq, _rq,
        _full((2, 128, 128)), _full((2, 128)), _full((2, 128, 128)),
        _full((2, 128)), _full((2, 128, 128)), _full((2, 128)),
    ],
    out_specs=_rq,
    out_shape=jax.ShapeDtypeStruct((NPQ, 128), jnp.float32),
)


def _layer2_body(x1, a0, a1, w1, b1, w2, b2, lw, lb, wla, wlb, bl, out):
    acc = None
    for j in range(2):
        h = x1[...] + (a0[...] if j == 0 else a1[...])
        t = _branch(h, w1[j], b1[j], w2[j], b2[j], lw[j], lb[j])
        acc = t if acc is None else acc + t
    out[...] = (
        jnp.dot(x1[...], wla[...], preferred_element_type=jnp.float32)
        + jnp.dot(acc, wlb[...], preferred_element_type=jnp.float32)
        + bl[...]
    )


_layer2 = pl.pallas_call(
    _layer2_body,
    grid=_GRID,
    in_specs=[
        _rq, _rq, _rq,
        _full((2, 128, 128)), _full((2, 128)), _full((2, 128, 128)),
        _full((2, 128)), _full((2, 128, 128)), _full((2, 128)),
        _full((128, 128)), _full((128, 128)), _full((128,)),
    ],
    out_specs=_rq,
    out_shape=jax.ShapeDtypeStruct((NPQ, 128), jnp.float32),
)


def _bd(w):
    return jnp.kron(jnp.eye(4, dtype=w.dtype), w)


def _bd2(w):
    return jnp.stack([_bd(w[0]), _bd(w[1])])


def _t4(b):
    return jnp.tile(b, (1, 4))


def _prep(adj):
    pad = E_PAD - E
    pidx = jnp.arange(pad, dtype=jnp.int32)
    s2 = 2 * jnp.concatenate([adj[0], pidx % N])
    dst = jnp.concatenate([adj[1], NP + (pidx & (TRASH - 1))])
    return jnp.stack([jnp.stack([s2, dst]),
                      jnp.stack([s2 + 1, dst])]).reshape(
                          NC, 2, E_PAD // 128, 128)


def kernel(adjs_0, adjs_1, embed, gin_w1, gin_b1, gin_w2, gin_b2, lin_w, lin_b, w_last, b_last):
    x0 = jnp.pad(embed, ((0, NP - N), (0, 0)))
    x0f = x0.reshape(2 * NP, F)
    adjs_0 = lax.optimization_barrier((adjs_0, x0f))[0]
    sd0 = _prep(adjs_0)
    adjs_1 = lax.optimization_barrier((adjs_1, sd0))[0]
    sd1 = _prep(adjs_1)

    def segsum(x, sd):
        return _segsum(x.reshape(2 * NP, F), sd).reshape(NPQ, 128)

    w1a, b1a = _bd2(gin_w1[0]), _t4(gin_b1[0])
    w2a, b2a = _bd2(gin_w2[0]), _t4(gin_b2[0])
    lwa, lba = _bd2(lin_w[0]), _t4(lin_b[0])
    w1b, b1b = _bd2(gin_w1[1]), _t4(gin_b1[1])
    w2b, b2b = _bd2(gin_w2[1]), _t4(gin_b2[1])
    lwb, lbb = _bd2(lin_w[1]), _t4(lin_b[1])
    wla, wlb = _bd(w_last[:D]), _bd(w_last[D:])
    blt = jnp.tile(b_last, 4)

    x0q = x0.reshape(NPQ, 128)
    hp0 = segsum(x0, sd0)
    hp1 = segsum(x0, sd1)
    x1 = _layer1(x0q, hp0, hp1, w1a, b1a, w2a, b2a, lwa, lba)
    x1n = x1.reshape(NP, D)
    hp0b = segsum(x1n, sd0)
    hp1b = segsum(x1n, sd1)
    y = _layer2(x1, hp0b, hp1b, w1b, b1b, w2b, b2b, lwb, lbb, wla, wlb, blt)
    return y.reshape(NP, D)[:N]
